# baseline (speedup 1.0000x reference)
import functools

import jax
import jax.numpy as jnp
from jax import lax
from jax.experimental import pallas as pl
from jax.experimental.pallas import tpu as pltpu
from jax.experimental.pallas import tpu_sc as plsc

N = 10000
E = 160000
D = 128
R = 8
NB = 4

NC = 2
NS = 16
NW = NC * NS
L = 16

CH = 320
NCHUNK = E // CH
WITER = -(-NCHUNK // NW)
NPT = 624
NGRP = CH // L
NSTREAM = 4
SB = CH // NSTREAM
NCHP = 512
WITER2 = NCHP // NW

_mesh = plsc.VectorSubcoreMesh(core_axis_name="c", subcore_axis_name="s")


def _cnt_body(dst_hbm, et_hbm, out_hbm, histv, dstv, etv):
    cid = lax.axis_index("c")
    sid = lax.axis_index("s")
    wid = sid * NC + cid

    def _zero(i, _):
        histv[pl.ds(i * L, L)] = jnp.zeros((L,), jnp.float32)
        return 0

    lax.fori_loop(0, (N * R) // L, _zero, 0)

    ones = jnp.ones((L,), jnp.float32)

    def _chunk(k, _):
        cidx = k * NW + wid

        @pl.when(cidx < NCHUNK)
        def _():
            base = cidx * CH
            pltpu.sync_copy(dst_hbm.at[pl.ds(base, CH)], dstv)
            pltpu.sync_copy(et_hbm.at[pl.ds(base, CH)], etv)
            for g in range(NGRP):
                d16 = dstv[pl.ds(g * L, L)]
                t16 = etv[pl.ds(g * L, L)]
                idx = d16 * R + t16
                plsc.addupdate_scatter(histv, [idx], ones)

        return 0

    lax.fori_loop(0, WITER, _chunk, 0)
    pltpu.sync_copy(histv, out_hbm.at[pl.ds(wid * (N * R), N * R)])


_cnt_kernel = pl.kernel(
    _cnt_body,
    out_type=jax.ShapeDtypeStruct((NW * N * R,), jnp.float32),
    mesh=_mesh,
    scratch_types=[
        pltpu.VMEM((N * R,), jnp.float32),
        pltpu.VMEM((CH,), jnp.int32),
        pltpu.VMEM((CH,), jnp.int32),
    ],
    compiler_params=pltpu.CompilerParams(needs_layout_passes=False),
)


def _inv_body(dst_hbm, et_hbm, inv_hbm, out_hbm, invv, dstv, etv, outv):
    cid = lax.axis_index("c")
    sid = lax.axis_index("s")
    wid = sid * NC + cid
    pltpu.sync_copy(inv_hbm, invv)

    def _chunk(k, _):
        cidx = k * NW + wid
        base = cidx * CH

        @pl.when(cidx < NCHUNK)
        def _():
            pltpu.sync_copy(dst_hbm.at[pl.ds(base, CH)], dstv)
            pltpu.sync_copy(et_hbm.at[pl.ds(base, CH)], etv)
            for g in range(NGRP):
                d16 = dstv[pl.ds(g * L, L)]
                t16 = etv[pl.ds(g * L, L)]
                outv[pl.ds(g * L, L)] = plsc.load_gather(invv, [d16 * R + t16])
            pltpu.sync_copy(outv, out_hbm.at[pl.ds(base, CH)])

        @pl.when(cidx >= NCHUNK)
        def _():
            for g in range(NGRP):
                outv[pl.ds(g * L, L)] = jnp.zeros((L,), jnp.float32)
            pltpu.sync_copy(outv, out_hbm.at[pl.ds(base, CH)])

        return 0

    lax.fori_loop(0, WITER, _chunk, 0)


_inv_kernel = pl.kernel(
    _inv_body,
    out_type=jax.ShapeDtypeStruct((NCHP * CH,), jnp.float32),
    mesh=_mesh,
    scratch_types=[
        pltpu.VMEM((N * R,), jnp.float32),
        pltpu.VMEM((CH,), jnp.int32),
        pltpu.VMEM((CH,), jnp.int32),
        pltpu.VMEM((CH,), jnp.float32),
    ],
    compiler_params=pltpu.CompilerParams(needs_layout_passes=False),
)


def _edge_body(y_hbm, src_hbm, dst_hbm, et_hbm, inve_hbm, out_hbm,
               srcv, dstv, etv, gidx, didx, fbuf, rows, agg_sh, sem):
    cid = lax.axis_index("c")
    sid = lax.axis_index("s")
    wid = sid * NC + cid

    def _zrow(zr, _):
        for j in range(D // L):
            rows[zr, pl.ds(j * L, L)] = jnp.zeros((L,), jnp.float32)
        return 0

    lax.fori_loop(0, 104, _zrow, 0)
    for q in range(NPT // 104):
        pltpu.sync_copy(rows.at[pl.ds(0, 104)],
                        agg_sh.at[pl.ds(sid * NPT + q * 104, 104)])

    @pl.when(sid == NS - 1)
    def _():
        pltpu.sync_copy(rows.at[pl.ds(0, N - NS * NPT)],
                        agg_sh.at[pl.ds(NS * NPT, N - NS * NPT)])

    plsc.subcore_barrier()

    def _chunk(k, _):
        cidx = k * NW + wid
        base = cidx * CH
        pltpu.sync_copy(src_hbm.at[pl.ds(base, CH)], srcv)
        pltpu.sync_copy(dst_hbm.at[pl.ds(base, CH)], dstv)
        pltpu.sync_copy(et_hbm.at[pl.ds(base, CH)], etv)
        pltpu.sync_copy(inve_hbm.at[pl.ds(base, CH)], fbuf)
        for g in range(NGRP):
            s16 = srcv[pl.ds(g * L, L)]
            t16 = etv[pl.ds(g * L, L)]
            d16 = dstv[pl.ds(g * L, L)]
            gidx[g // (NGRP // NSTREAM),
                 pl.ds((g % (NGRP // NSTREAM)) * L, L)] = s16 * R + t16
            didx[g // (NGRP // NSTREAM),
                 pl.ds((g % (NGRP // NSTREAM)) * L, L)] = d16
        cps = [pltpu.async_copy(y_hbm.at[gidx.at[j]],
                                rows.at[pl.ds(j * SB, SB)], sem)
               for j in range(NSTREAM)]
        for cp in cps:
            cp.wait()

        @plsc.parallel_loop(0, CH, 1, unroll=8)
        def _scale(e):
            fv = plsc.load_gather(fbuf, [jnp.full((L,), e, jnp.int32)])
            for j in range(D // L):
                rows[e, pl.ds(j * L, L)] = rows[e, pl.ds(j * L, L)] * fv

        for j in range(NSTREAM):
            pltpu.sync_copy(rows.at[pl.ds(j * SB, SB)],
                            agg_sh.at[didx.at[j]], add=True)
        return 0

    lax.fori_loop(0, WITER2, _chunk, 0)
    plsc.subcore_barrier()
    pltpu.sync_copy(agg_sh.at[pl.ds(sid * NPT, NPT)],
                    out_hbm.at[pl.ds(cid * N + sid * NPT, NPT)])

    @pl.when(sid == NS - 1)
    def _():
        pltpu.sync_copy(agg_sh.at[pl.ds(NS * NPT, N - NS * NPT)],
                        out_hbm.at[pl.ds(cid * N + NS * NPT, N - NS * NPT)])


_edge_kernel = pl.kernel(
    _edge_body,
    out_type=jax.ShapeDtypeStruct((NC * N, D), jnp.float32),
    mesh=_mesh,
    scratch_types=[
        pltpu.VMEM((CH,), jnp.int32),
        pltpu.VMEM((CH,), jnp.int32),
        pltpu.VMEM((CH,), jnp.int32),
        pltpu.VMEM((NSTREAM, SB), jnp.int32),
        pltpu.VMEM((NSTREAM, SB), jnp.int32),
        pltpu.VMEM((CH,), jnp.float32),
        pltpu.VMEM((CH, D), jnp.float32),
        pltpu.VMEM_SHARED((N, D), jnp.float32),
        pltpu.SemaphoreType.DMA,
    ],
    compiler_params=pltpu.CompilerParams(needs_layout_passes=False),
)


NBLK = 25
BR = N // NBLK


def _wcat(comp_ref, bases_ref, r):
    c = comp_ref[...]
    b = bases_ref[...]
    w = c[r, 0] * b[0]
    for q in range(1, NB):
        w = w + c[r, q] * b[q]
    return w


def _prep_body(x_ref, imp_ref, cnt_ref, comp_ref, bases_ref,
               x0_ref, y_ref, inv_ref):
    x0 = x_ref[...] * imp_ref[...]
    x0_ref[...] = x0
    for r in range(R):
        w = _wcat(comp_ref, bases_ref, r)
        y_ref[:, r * D:(r + 1) * D] = jnp.dot(
            x0, w, preferred_element_type=jnp.float32)
    cnt = jnp.sum(cnt_ref[...], axis=0)
    inv_ref[...] = 1.0 / jnp.maximum(cnt, 1.0)


_prep_kernel = pl.pallas_call(
    _prep_body,
    grid=(NBLK,),
    in_specs=[
        pl.BlockSpec((BR, D), lambda i: (i, 0)),
        pl.BlockSpec((BR, 1), lambda i: (i, 0)),
        pl.BlockSpec((NW, BR, R), lambda i: (0, i, 0)),
        pl.BlockSpec((R, NB), lambda i: (0, 0)),
        pl.BlockSpec((NB, D, D), lambda i: (0, 0, 0)),
    ],
    out_specs=[
        pl.BlockSpec((BR, D), lambda i: (i, 0)),
        pl.BlockSpec((BR, R * D), lambda i: (i, 0)),
        pl.BlockSpec((BR, R), lambda i: (i, 0)),
    ],
    out_shape=[
        jax.ShapeDtypeStruct((N, D), jnp.float32),
        jax.ShapeDtypeStruct((N, R * D), jnp.float32),
        jax.ShapeDtypeStruct((N, R), jnp.float32),
    ],
)


def _layer_tail(aggp, xin, root, bias, gamma, beta, mean, var):
    agg = aggp[0] + aggp[1]
    h = agg + jnp.dot(xin, root, preferred_element_type=jnp.float32) + bias
    h = jnp.maximum(h, 0.0)
    return (h - mean) * lax.rsqrt(var + 1e-5) * gamma + beta


def _mid_body(aggp_ref, x0_ref, root_ref, bias_ref, g_ref, b_ref, m_ref,
              v_ref, comp_ref, bases_ref, x1_ref, y_ref):
    x1 = _layer_tail(aggp_ref[...], x0_ref[...], root_ref[...], bias_ref[...],
                     g_ref[...], b_ref[...], m_ref[...], v_ref[...])
    x1_ref[...] = x1
    for r in range(R):
        w = _wcat(comp_ref, bases_ref, r)
        y_ref[:, r * D:(r + 1) * D] = jnp.dot(
            x1, w, preferred_element_type=jnp.float32)


_mid_kernel = pl.pallas_call(
    _mid_body,
    grid=(NBLK,),
    in_specs=[
        pl.BlockSpec((NC, BR, D), lambda i: (0, i, 0)),
        pl.BlockSpec((BR, D), lambda i: (i, 0)),
        pl.BlockSpec((D, D), lambda i: (0, 0)),
        pl.BlockSpec((1, D), lambda i: (0, 0)),
        pl.BlockSpec((1, D), lambda i: (0, 0)),
        pl.BlockSpec((1, D), lambda i: (0, 0)),
        pl.BlockSpec((1, D), lambda i: (0, 0)),
        pl.BlockSpec((1, D), lambda i: (0, 0)),
        pl.BlockSpec((R, NB), lambda i: (0, 0)),
        pl.BlockSpec((NB, D, D), lambda i: (0, 0, 0)),
    ],
    out_specs=[
        pl.BlockSpec((BR, D), lambda i: (i, 0)),
        pl.BlockSpec((BR, R * D), lambda i: (i, 0)),
    ],
    out_shape=[
        jax.ShapeDtypeStruct((N, D), jnp.float32),
        jax.ShapeDtypeStruct((N, R * D), jnp.float32),
    ],
)


def _final_body(aggp_ref, x1_ref, root_ref, bias_ref, g_ref, b_ref, m_ref,
                v_ref, out_ref):
    out_ref[...] = _layer_tail(
        aggp_ref[...], x1_ref[...], root_ref[...], bias_ref[...],
        g_ref[...], b_ref[...], m_ref[...], v_ref[...])


_final_kernel = pl.pallas_call(
    _final_body,
    grid=(NBLK,),
    in_specs=[
        pl.BlockSpec((NC, BR, D), lambda i: (0, i, 0)),
        pl.BlockSpec((BR, D), lambda i: (i, 0)),
        pl.BlockSpec((D, D), lambda i: (0, 0)),
        pl.BlockSpec((1, D), lambda i: (0, 0)),
        pl.BlockSpec((1, D), lambda i: (0, 0)),
        pl.BlockSpec((1, D), lambda i: (0, 0)),
        pl.BlockSpec((1, D), lambda i: (0, 0)),
        pl.BlockSpec((1, D), lambda i: (0, 0)),
    ],
    out_specs=pl.BlockSpec((BR, D), lambda i: (i, 0)),
    out_shape=jax.ShapeDtypeStruct((N, D), jnp.float32),
)


@jax.jit
def kernel(X, X_importance, edge_index, edge_type, index,
           comp0, bases0, root0, bias0, gamma0, beta0, mean0, var0,
           comp1, bases1, root1, bias1, gamma1, beta1, mean1, var1):
    del index
    epad = NCHP * CH - E
    zi = jnp.zeros((epad,), jnp.int32)
    srcp = jnp.concatenate([edge_index[0], zi])
    dstp = jnp.concatenate([edge_index[1], zi])
    etp = jnp.concatenate([edge_type, zi])

    cnt = _cnt_kernel(dstp, etp).reshape(NW, N, R)
    x0, y0, inv = _prep_kernel(X, X_importance, cnt, comp0, bases0)
    invp = _inv_kernel(dstp, etp, inv.reshape(N * R))

    aggp0 = _edge_kernel(y0.reshape(N * R, D), srcp, dstp, etp,
                         invp).reshape(NC, N, D)
    x1, y1 = _mid_kernel(aggp0, x0, root0, bias0.reshape(1, D),
                         gamma0.reshape(1, D), beta0.reshape(1, D),
                         mean0.reshape(1, D), var0.reshape(1, D),
                         comp1, bases1)
    aggp1 = _edge_kernel(y1.reshape(N * R, D), srcp, dstp, etp,
                         invp).reshape(NC, N, D)
    out = _final_kernel(aggp1, x1, root1, bias1.reshape(1, D),
                        gamma1.reshape(1, D), beta1.reshape(1, D),
                        mean1.reshape(1, D), var1.reshape(1, D))
    return out


# speedup vs baseline: 1.4123x; 1.4123x over previous
import functools

import jax
import jax.numpy as jnp
from jax import lax
from jax.experimental import pallas as pl
from jax.experimental.pallas import tpu as pltpu
from jax.experimental.pallas import tpu_sc as plsc

N = 10000
E = 160000
D = 128
R = 8
NB = 4

NC = 2
NS = 16
NW = NC * NS
L = 16

CH = 320
NCHUNK = E // CH
WITER = -(-NCHUNK // NW)
NPT = 624
NGRP = CH // L
NSTREAM = 4
SB = CH // NSTREAM
NCHP = 512
WITER2 = NCHP // NW

_mesh = plsc.VectorSubcoreMesh(core_axis_name="c", subcore_axis_name="s")


def _cnt_body(dst_hbm, et_hbm, out_hbm, histv, dstv, etv):
    cid = lax.axis_index("c")
    sid = lax.axis_index("s")
    wid = sid * NC + cid

    def _zero(i, _):
        histv[pl.ds(i * L, L)] = jnp.zeros((L,), jnp.float32)
        return 0

    lax.fori_loop(0, (N * R) // L, _zero, 0)

    ones = jnp.ones((L,), jnp.float32)

    def _chunk(k, _):
        cidx = k * NW + wid

        @pl.when(cidx < NCHUNK)
        def _():
            base = cidx * CH
            pltpu.sync_copy(dst_hbm.at[pl.ds(base, CH)], dstv)
            pltpu.sync_copy(et_hbm.at[pl.ds(base, CH)], etv)
            for g in range(NGRP):
                d16 = dstv[pl.ds(g * L, L)]
                t16 = etv[pl.ds(g * L, L)]
                idx = d16 * R + t16
                plsc.addupdate_scatter(histv, [idx], ones)

        return 0

    lax.fori_loop(0, WITER, _chunk, 0)
    pltpu.sync_copy(histv, out_hbm.at[pl.ds(wid * (N * R), N * R)])


_cnt_kernel = pl.kernel(
    _cnt_body,
    out_type=jax.ShapeDtypeStruct((NW * N * R,), jnp.float32),
    mesh=_mesh,
    scratch_types=[
        pltpu.VMEM((N * R,), jnp.float32),
        pltpu.VMEM((CH,), jnp.int32),
        pltpu.VMEM((CH,), jnp.int32),
    ],
    compiler_params=pltpu.CompilerParams(needs_layout_passes=False),
)


def _inv_body(dst_hbm, et_hbm, inv_hbm, out_hbm, invv, dstv, etv, outv):
    cid = lax.axis_index("c")
    sid = lax.axis_index("s")
    wid = sid * NC + cid
    pltpu.sync_copy(inv_hbm, invv)

    def _chunk(k, _):
        cidx = k * NW + wid
        base = cidx * CH

        @pl.when(cidx < NCHUNK)
        def _():
            pltpu.sync_copy(dst_hbm.at[pl.ds(base, CH)], dstv)
            pltpu.sync_copy(et_hbm.at[pl.ds(base, CH)], etv)
            for g in range(NGRP):
                d16 = dstv[pl.ds(g * L, L)]
                t16 = etv[pl.ds(g * L, L)]
                outv[pl.ds(g * L, L)] = plsc.load_gather(invv, [d16 * R + t16])
            pltpu.sync_copy(outv, out_hbm.at[pl.ds(base, CH)])

        @pl.when(cidx >= NCHUNK)
        def _():
            for g in range(NGRP):
                outv[pl.ds(g * L, L)] = jnp.zeros((L,), jnp.float32)
            pltpu.sync_copy(outv, out_hbm.at[pl.ds(base, CH)])

        return 0

    lax.fori_loop(0, WITER, _chunk, 0)


_inv_kernel = pl.kernel(
    _inv_body,
    out_type=jax.ShapeDtypeStruct((NCHP * CH,), jnp.float32),
    mesh=_mesh,
    scratch_types=[
        pltpu.VMEM((N * R,), jnp.float32),
        pltpu.VMEM((CH,), jnp.int32),
        pltpu.VMEM((CH,), jnp.int32),
        pltpu.VMEM((CH,), jnp.float32),
    ],
    compiler_params=pltpu.CompilerParams(needs_layout_passes=False),
)


def _edge_body(y_hbm, src_hbm, dst_hbm, et_hbm, inve_hbm, out_hbm,
               srcv, dstv, etv, gidx, didx, fbuf, rows, agg_sh, sem):
    cid = lax.axis_index("c")
    sid = lax.axis_index("s")
    wid = sid * NC + cid

    def _zrow(zr, _):
        for j in range(D // L):
            rows[zr, pl.ds(j * L, L)] = jnp.zeros((L,), jnp.float32)
        return 0

    lax.fori_loop(0, 104, _zrow, 0)
    for q in range(NPT // 104):
        pltpu.sync_copy(rows.at[pl.ds(0, 104)],
                        agg_sh.at[pl.ds(sid * NPT + q * 104, 104)])

    @pl.when(sid == NS - 1)
    def _():
        pltpu.sync_copy(rows.at[pl.ds(0, N - NS * NPT)],
                        agg_sh.at[pl.ds(NS * NPT, N - NS * NPT)])

    plsc.subcore_barrier()

    def _chunk(k, _):
        cidx = k * NW + wid
        base = cidx * CH
        pltpu.sync_copy(src_hbm.at[pl.ds(base, CH)], srcv)
        pltpu.sync_copy(dst_hbm.at[pl.ds(base, CH)], dstv)
        pltpu.sync_copy(et_hbm.at[pl.ds(base, CH)], etv)
        pltpu.sync_copy(inve_hbm.at[pl.ds(base, CH)], fbuf)
        for g in range(NGRP):
            s16 = srcv[pl.ds(g * L, L)]
            t16 = etv[pl.ds(g * L, L)]
            d16 = dstv[pl.ds(g * L, L)]
            gidx[g // (NGRP // NSTREAM),
                 pl.ds((g % (NGRP // NSTREAM)) * L, L)] = s16 * R + t16
            didx[g // (NGRP // NSTREAM),
                 pl.ds((g % (NGRP // NSTREAM)) * L, L)] = d16
        cps = [pltpu.async_copy(y_hbm.at[gidx.at[j]],
                                rows.at[pl.ds(j * SB, SB)], sem)
               for j in range(NSTREAM)]
        for cp in cps:
            cp.wait()

        @plsc.parallel_loop(0, CH, 1, unroll=8)
        def _scale(e):
            fv = plsc.load_gather(fbuf, [jnp.full((L,), e, jnp.int32)])
            for j in range(D // L):
                rows[e, pl.ds(j * L, L)] = rows[e, pl.ds(j * L, L)] * fv

        for j in range(NSTREAM):
            pltpu.sync_copy(rows.at[pl.ds(j * SB, SB)],
                            agg_sh.at[didx.at[j]], add=True)
        return 0

    lax.fori_loop(0, WITER2, _chunk, 0)
    plsc.subcore_barrier()
    pltpu.sync_copy(agg_sh.at[pl.ds(sid * NPT, NPT)],
                    out_hbm.at[pl.ds(cid * N + sid * NPT, NPT)])

    @pl.when(sid == NS - 1)
    def _():
        pltpu.sync_copy(agg_sh.at[pl.ds(NS * NPT, N - NS * NPT)],
                        out_hbm.at[pl.ds(cid * N + NS * NPT, N - NS * NPT)])


_edge_kernel = pl.kernel(
    _edge_body,
    out_type=jax.ShapeDtypeStruct((NC * N, D), jnp.float32),
    mesh=_mesh,
    scratch_types=[
        pltpu.VMEM((CH,), jnp.int32),
        pltpu.VMEM((CH,), jnp.int32),
        pltpu.VMEM((CH,), jnp.int32),
        pltpu.VMEM((NSTREAM, SB), jnp.int32),
        pltpu.VMEM((NSTREAM, SB), jnp.int32),
        pltpu.VMEM((CH,), jnp.float32),
        pltpu.VMEM((CH, D), jnp.float32),
        pltpu.VMEM_SHARED((N, D), jnp.float32),
        pltpu.SemaphoreType.DMA,
    ],
    compiler_params=pltpu.CompilerParams(needs_layout_passes=False),
)


NBLK = 25
BR = N // NBLK


def _wcat(comp_ref, bases_ref, r):
    c = comp_ref[...]
    b = bases_ref[...]
    w = c[r, 0] * b[0]
    for q in range(1, NB):
        w = w + c[r, q] * b[q]
    return w


def _prep_body(x_ref, imp_ref, cnt_ref, comp_ref, bases_ref,
               x0_ref, y_ref, inv_ref):
    x0 = x_ref[...] * imp_ref[...]
    x0_ref[...] = x0
    for r in range(R):
        w = _wcat(comp_ref, bases_ref, r)
        y_ref[:, r * D:(r + 1) * D] = jnp.dot(
            x0, w, preferred_element_type=jnp.float32)
    cnt = jnp.sum(cnt_ref[...], axis=0)
    inv_ref[...] = 1.0 / jnp.maximum(cnt, 1.0)


_prep_kernel = pl.pallas_call(
    _prep_body,
    grid=(NBLK,),
    in_specs=[
        pl.BlockSpec((BR, D), lambda i: (i, 0)),
        pl.BlockSpec((BR, 1), lambda i: (i, 0)),
        pl.BlockSpec((NW, BR, R), lambda i: (0, i, 0)),
        pl.BlockSpec((R, NB), lambda i: (0, 0)),
        pl.BlockSpec((NB, D, D), lambda i: (0, 0, 0)),
    ],
    out_specs=[
        pl.BlockSpec((BR, D), lambda i: (i, 0)),
        pl.BlockSpec((BR, R * D), lambda i: (i, 0)),
        pl.BlockSpec((BR, R), lambda i: (i, 0)),
    ],
    out_shape=[
        jax.ShapeDtypeStruct((N, D), jnp.float32),
        jax.ShapeDtypeStruct((N, R * D), jnp.float32),
        jax.ShapeDtypeStruct((N, R), jnp.float32),
    ],
)


def _layer_tail(aggp, xin, root, bias, gamma, beta, mean, var):
    agg = aggp[0] + aggp[1]
    h = agg + jnp.dot(xin, root, preferred_element_type=jnp.float32) + bias
    h = jnp.maximum(h, 0.0)
    return (h - mean) * lax.rsqrt(var + 1e-5) * gamma + beta


def _mid_body(aggp_ref, x0_ref, root_ref, bias_ref, g_ref, b_ref, m_ref,
              v_ref, comp_ref, bases_ref, x1_ref, y_ref):
    x1 = _layer_tail(aggp_ref[...], x0_ref[...], root_ref[...], bias_ref[...],
                     g_ref[...], b_ref[...], m_ref[...], v_ref[...])
    x1_ref[...] = x1
    for r in range(R):
        w = _wcat(comp_ref, bases_ref, r)
        y_ref[:, r * D:(r + 1) * D] = jnp.dot(
            x1, w, preferred_element_type=jnp.float32)


_mid_kernel = pl.pallas_call(
    _mid_body,
    grid=(NBLK,),
    in_specs=[
        pl.BlockSpec((NC, BR, D), lambda i: (0, i, 0)),
        pl.BlockSpec((BR, D), lambda i: (i, 0)),
        pl.BlockSpec((D, D), lambda i: (0, 0)),
        pl.BlockSpec((1, D), lambda i: (0, 0)),
        pl.BlockSpec((1, D), lambda i: (0, 0)),
        pl.BlockSpec((1, D), lambda i: (0, 0)),
        pl.BlockSpec((1, D), lambda i: (0, 0)),
        pl.BlockSpec((1, D), lambda i: (0, 0)),
        pl.BlockSpec((R, NB), lambda i: (0, 0)),
        pl.BlockSpec((NB, D, D), lambda i: (0, 0, 0)),
    ],
    out_specs=[
        pl.BlockSpec((BR, D), lambda i: (i, 0)),
        pl.BlockSpec((BR, R * D), lambda i: (i, 0)),
    ],
    out_shape=[
        jax.ShapeDtypeStruct((N, D), jnp.float32),
        jax.ShapeDtypeStruct((N, R * D), jnp.float32),
    ],
)


def _final_body(aggp_ref, x1_ref, root_ref, bias_ref, g_ref, b_ref, m_ref,
                v_ref, out_ref):
    out_ref[...] = _layer_tail(
        aggp_ref[...], x1_ref[...], root_ref[...], bias_ref[...],
        g_ref[...], b_ref[...], m_ref[...], v_ref[...])


_final_kernel = pl.pallas_call(
    _final_body,
    grid=(NBLK,),
    in_specs=[
        pl.BlockSpec((NC, BR, D), lambda i: (0, i, 0)),
        pl.BlockSpec((BR, D), lambda i: (i, 0)),
        pl.BlockSpec((D, D), lambda i: (0, 0)),
        pl.BlockSpec((1, D), lambda i: (0, 0)),
        pl.BlockSpec((1, D), lambda i: (0, 0)),
        pl.BlockSpec((1, D), lambda i: (0, 0)),
        pl.BlockSpec((1, D), lambda i: (0, 0)),
        pl.BlockSpec((1, D), lambda i: (0, 0)),
    ],
    out_specs=pl.BlockSpec((BR, D), lambda i: (i, 0)),
    out_shape=jax.ShapeDtypeStruct((N, D), jnp.float32),
)


@jax.jit
def kernel(X, X_importance, edge_index, edge_type, index,
           comp0, bases0, root0, bias0, gamma0, beta0, mean0, var0,
           comp1, bases1, root1, bias1, gamma1, beta1, mean1, var1):
    del index
    epad = NCHP * CH - E
    pad_ids = jnp.arange(epad, dtype=jnp.int32)
    srcp = jnp.concatenate([edge_index[0], pad_ids % N])
    dstp = jnp.concatenate([edge_index[1], pad_ids % N])
    etp = jnp.concatenate([edge_type, pad_ids % R])

    cnt = _cnt_kernel(dstp, etp).reshape(NW, N, R)
    x0, y0, inv = _prep_kernel(X, X_importance, cnt, comp0, bases0)
    invp = _inv_kernel(dstp, etp, inv.reshape(N * R))

    aggp0 = _edge_kernel(y0.reshape(N * R, D), srcp, dstp, etp,
                         invp).reshape(NC, N, D)
    x1, y1 = _mid_kernel(aggp0, x0, root0, bias0.reshape(1, D),
                         gamma0.reshape(1, D), beta0.reshape(1, D),
                         mean0.reshape(1, D), var0.reshape(1, D),
                         comp1, bases1)
    aggp1 = _edge_kernel(y1.reshape(N * R, D), srcp, dstp, etp,
                         invp).reshape(NC, N, D)
    out = _final_kernel(aggp1, x1, root1, bias1.reshape(1, D),
                        gamma1.reshape(1, D), beta1.reshape(1, D),
                        mean1.reshape(1, D), var1.reshape(1, D))
    return out


# speedup vs baseline: 1.5272x; 1.0814x over previous
import functools

import jax
import jax.numpy as jnp
from jax import lax
from jax.experimental import pallas as pl
from jax.experimental.pallas import tpu as pltpu
from jax.experimental.pallas import tpu_sc as plsc

N = 10000
E = 160000
D = 128
R = 8
NB = 4

NC = 2
NS = 16
NW = NC * NS
L = 16

CH = 320
NCHUNK = E // CH
WITER = -(-NCHUNK // NW)
NPT = 624
NGRP = CH // L
NSTREAM = 4
SB = CH // NSTREAM
NCHP = 512
WITER2 = NCHP // NW

_mesh = plsc.VectorSubcoreMesh(core_axis_name="c", subcore_axis_name="s")


def _cnt_body(dst_hbm, et_hbm, out_hbm, histv, dstv, etv):
    cid = lax.axis_index("c")
    sid = lax.axis_index("s")
    wid = sid * NC + cid

    def _zero(i, _):
        histv[pl.ds(i * L, L)] = jnp.zeros((L,), jnp.float32)
        return 0

    lax.fori_loop(0, (N * R) // L, _zero, 0)

    ones = jnp.ones((L,), jnp.float32)

    def _chunk(k, _):
        cidx = k * NW + wid

        @pl.when(cidx < NCHUNK)
        def _():
            base = cidx * CH
            pltpu.sync_copy(dst_hbm.at[pl.ds(base, CH)], dstv)
            pltpu.sync_copy(et_hbm.at[pl.ds(base, CH)], etv)
            for g in range(NGRP):
                d16 = dstv[pl.ds(g * L, L)]
                t16 = etv[pl.ds(g * L, L)]
                idx = d16 * R + t16
                plsc.addupdate_scatter(histv, [idx], ones)

        return 0

    lax.fori_loop(0, WITER, _chunk, 0)
    pltpu.sync_copy(histv, out_hbm.at[pl.ds(wid * (N * R), N * R)])


_cnt_kernel = pl.kernel(
    _cnt_body,
    out_type=jax.ShapeDtypeStruct((NW * N * R,), jnp.float32),
    mesh=_mesh,
    scratch_types=[
        pltpu.VMEM((N * R,), jnp.float32),
        pltpu.VMEM((CH,), jnp.int32),
        pltpu.VMEM((CH,), jnp.int32),
    ],
    compiler_params=pltpu.CompilerParams(needs_layout_passes=False),
)


def _inv_body(dst_hbm, et_hbm, inv_hbm, out_hbm, invv, dstv, etv, outv):
    cid = lax.axis_index("c")
    sid = lax.axis_index("s")
    wid = sid * NC + cid
    pltpu.sync_copy(inv_hbm, invv)

    def _chunk(k, _):
        cidx = k * NW + wid
        base = cidx * CH

        @pl.when(cidx < NCHUNK)
        def _():
            pltpu.sync_copy(dst_hbm.at[pl.ds(base, CH)], dstv)
            pltpu.sync_copy(et_hbm.at[pl.ds(base, CH)], etv)
            for g in range(NGRP):
                d16 = dstv[pl.ds(g * L, L)]
                t16 = etv[pl.ds(g * L, L)]
                outv[pl.ds(g * L, L)] = plsc.load_gather(invv, [d16 * R + t16])
            pltpu.sync_copy(outv, out_hbm.at[pl.ds(base, CH)])

        @pl.when(cidx >= NCHUNK)
        def _():
            for g in range(NGRP):
                outv[pl.ds(g * L, L)] = jnp.zeros((L,), jnp.float32)
            pltpu.sync_copy(outv, out_hbm.at[pl.ds(base, CH)])

        return 0

    lax.fori_loop(0, WITER, _chunk, 0)


_inv_kernel = pl.kernel(
    _inv_body,
    out_type=jax.ShapeDtypeStruct((NCHP * CH,), jnp.float32),
    mesh=_mesh,
    scratch_types=[
        pltpu.VMEM((N * R,), jnp.float32),
        pltpu.VMEM((CH,), jnp.int32),
        pltpu.VMEM((CH,), jnp.int32),
        pltpu.VMEM((CH,), jnp.float32),
    ],
    compiler_params=pltpu.CompilerParams(needs_layout_passes=False),
)


def _edge_body(y_hbm, src_hbm, dst_hbm, et_hbm, inve_hbm, out_hbm,
               srcv, dstv, etv, gidx, didx, fbuf, rows, agg_sh, sem, sem2):
    cid = lax.axis_index("c")
    sid = lax.axis_index("s")
    wid = sid * NC + cid

    def _zrow(zr, _):
        for j in range(D // L):
            rows[zr, pl.ds(j * L, L)] = jnp.zeros((L,), jnp.float32)
        return 0

    lax.fori_loop(0, 104, _zrow, 0)
    for q in range(NPT // 104):
        pltpu.sync_copy(rows.at[pl.ds(0, 104)],
                        agg_sh.at[pl.ds(sid * NPT + q * 104, 104)])

    @pl.when(sid == NS - 1)
    def _():
        pltpu.sync_copy(rows.at[pl.ds(0, N - NS * NPT)],
                        agg_sh.at[pl.ds(NS * NPT, N - NS * NPT)])

    plsc.subcore_barrier()

    def _wait_scatter():
        for j in range(NSTREAM):
            pltpu.make_async_copy(rows.at[pl.ds(j * SB, SB)],
                                  agg_sh.at[didx.at[j]], sem2).wait()

    def _chunk(k, _):
        cidx = k * NW + wid
        base = cidx * CH
        pltpu.sync_copy(src_hbm.at[pl.ds(base, CH)], srcv)
        pltpu.sync_copy(dst_hbm.at[pl.ds(base, CH)], dstv)
        pltpu.sync_copy(et_hbm.at[pl.ds(base, CH)], etv)
        pltpu.sync_copy(inve_hbm.at[pl.ds(base, CH)], fbuf)

        @pl.when(k > 0)
        def _():
            _wait_scatter()

        for g in range(NGRP):
            s16 = srcv[pl.ds(g * L, L)]
            t16 = etv[pl.ds(g * L, L)]
            d16 = dstv[pl.ds(g * L, L)]
            gidx[g // (NGRP // NSTREAM),
                 pl.ds((g % (NGRP // NSTREAM)) * L, L)] = s16 * R + t16
            didx[g // (NGRP // NSTREAM),
                 pl.ds((g % (NGRP // NSTREAM)) * L, L)] = d16
        cps = [pltpu.async_copy(y_hbm.at[gidx.at[j]],
                                rows.at[pl.ds(j * SB, SB)], sem)
               for j in range(NSTREAM)]
        for cp in cps:
            cp.wait()

        @plsc.parallel_loop(0, CH, 1, unroll=8)
        def _scale(e):
            fv = plsc.load_gather(fbuf, [jnp.full((L,), e, jnp.int32)])
            for j in range(D // L):
                rows[e, pl.ds(j * L, L)] = rows[e, pl.ds(j * L, L)] * fv

        for j in range(NSTREAM):
            pltpu.async_copy(rows.at[pl.ds(j * SB, SB)],
                             agg_sh.at[didx.at[j]], sem2, add=True)
        return 0

    lax.fori_loop(0, WITER2, _chunk, 0)
    _wait_scatter()
    plsc.subcore_barrier()
    pltpu.sync_copy(agg_sh.at[pl.ds(sid * NPT, NPT)],
                    out_hbm.at[pl.ds(cid * N + sid * NPT, NPT)])

    @pl.when(sid == NS - 1)
    def _():
        pltpu.sync_copy(agg_sh.at[pl.ds(NS * NPT, N - NS * NPT)],
                        out_hbm.at[pl.ds(cid * N + NS * NPT, N - NS * NPT)])


_edge_kernel = pl.kernel(
    _edge_body,
    out_type=jax.ShapeDtypeStruct((NC * N, D), jnp.float32),
    mesh=_mesh,
    scratch_types=[
        pltpu.VMEM((CH,), jnp.int32),
        pltpu.VMEM((CH,), jnp.int32),
        pltpu.VMEM((CH,), jnp.int32),
        pltpu.VMEM((NSTREAM, SB), jnp.int32),
        pltpu.VMEM((NSTREAM, SB), jnp.int32),
        pltpu.VMEM((CH,), jnp.float32),
        pltpu.VMEM((CH, D), jnp.float32),
        pltpu.VMEM_SHARED((N, D), jnp.float32),
        pltpu.SemaphoreType.DMA,
        pltpu.SemaphoreType.DMA,
    ],
    compiler_params=pltpu.CompilerParams(needs_layout_passes=False),
)


NBLK = 25
BR = N // NBLK


def _wcat(comp_ref, bases_ref, r):
    c = comp_ref[...]
    b = bases_ref[...]
    w = c[r, 0] * b[0]
    for q in range(1, NB):
        w = w + c[r, q] * b[q]
    return w


def _prep_body(x_ref, imp_ref, cnt_ref, comp_ref, bases_ref,
               x0_ref, y_ref, inv_ref):
    x0 = x_ref[...] * imp_ref[...]
    x0_ref[...] = x0
    for r in range(R):
        w = _wcat(comp_ref, bases_ref, r)
        y_ref[:, r * D:(r + 1) * D] = jnp.dot(
            x0, w, preferred_element_type=jnp.float32)
    cnt = jnp.sum(cnt_ref[...], axis=0)
    inv_ref[...] = 1.0 / jnp.maximum(cnt, 1.0)


_prep_kernel = pl.pallas_call(
    _prep_body,
    grid=(NBLK,),
    in_specs=[
        pl.BlockSpec((BR, D), lambda i: (i, 0)),
        pl.BlockSpec((BR, 1), lambda i: (i, 0)),
        pl.BlockSpec((NW, BR, R), lambda i: (0, i, 0)),
        pl.BlockSpec((R, NB), lambda i: (0, 0)),
        pl.BlockSpec((NB, D, D), lambda i: (0, 0, 0)),
    ],
    out_specs=[
        pl.BlockSpec((BR, D), lambda i: (i, 0)),
        pl.BlockSpec((BR, R * D), lambda i: (i, 0)),
        pl.BlockSpec((BR, R), lambda i: (i, 0)),
    ],
    out_shape=[
        jax.ShapeDtypeStruct((N, D), jnp.float32),
        jax.ShapeDtypeStruct((N, R * D), jnp.float32),
        jax.ShapeDtypeStruct((N, R), jnp.float32),
    ],
)


def _layer_tail(aggp, xin, root, bias, gamma, beta, mean, var):
    agg = aggp[0] + aggp[1]
    h = agg + jnp.dot(xin, root, preferred_element_type=jnp.float32) + bias
    h = jnp.maximum(h, 0.0)
    return (h - mean) * lax.rsqrt(var + 1e-5) * gamma + beta


def _mid_body(aggp_ref, x0_ref, root_ref, bias_ref, g_ref, b_ref, m_ref,
              v_ref, comp_ref, bases_ref, x1_ref, y_ref):
    x1 = _layer_tail(aggp_ref[...], x0_ref[...], root_ref[...], bias_ref[...],
                     g_ref[...], b_ref[...], m_ref[...], v_ref[...])
    x1_ref[...] = x1
    for r in range(R):
        w = _wcat(comp_ref, bases_ref, r)
        y_ref[:, r * D:(r + 1) * D] = jnp.dot(
            x1, w, preferred_element_type=jnp.float32)


_mid_kernel = pl.pallas_call(
    _mid_body,
    grid=(NBLK,),
    in_specs=[
        pl.BlockSpec((NC, BR, D), lambda i: (0, i, 0)),
        pl.BlockSpec((BR, D), lambda i: (i, 0)),
        pl.BlockSpec((D, D), lambda i: (0, 0)),
        pl.BlockSpec((1, D), lambda i: (0, 0)),
        pl.BlockSpec((1, D), lambda i: (0, 0)),
        pl.BlockSpec((1, D), lambda i: (0, 0)),
        pl.BlockSpec((1, D), lambda i: (0, 0)),
        pl.BlockSpec((1, D), lambda i: (0, 0)),
        pl.BlockSpec((R, NB), lambda i: (0, 0)),
        pl.BlockSpec((NB, D, D), lambda i: (0, 0, 0)),
    ],
    out_specs=[
        pl.BlockSpec((BR, D), lambda i: (i, 0)),
        pl.BlockSpec((BR, R * D), lambda i: (i, 0)),
    ],
    out_shape=[
        jax.ShapeDtypeStruct((N, D), jnp.float32),
        jax.ShapeDtypeStruct((N, R * D), jnp.float32),
    ],
)


def _final_body(aggp_ref, x1_ref, root_ref, bias_ref, g_ref, b_ref, m_ref,
                v_ref, out_ref):
    out_ref[...] = _layer_tail(
        aggp_ref[...], x1_ref[...], root_ref[...], bias_ref[...],
        g_ref[...], b_ref[...], m_ref[...], v_ref[...])


_final_kernel = pl.pallas_call(
    _final_body,
    grid=(NBLK,),
    in_specs=[
        pl.BlockSpec((NC, BR, D), lambda i: (0, i, 0)),
        pl.BlockSpec((BR, D), lambda i: (i, 0)),
        pl.BlockSpec((D, D), lambda i: (0, 0)),
        pl.BlockSpec((1, D), lambda i: (0, 0)),
        pl.BlockSpec((1, D), lambda i: (0, 0)),
        pl.BlockSpec((1, D), lambda i: (0, 0)),
        pl.BlockSpec((1, D), lambda i: (0, 0)),
        pl.BlockSpec((1, D), lambda i: (0, 0)),
    ],
    out_specs=pl.BlockSpec((BR, D), lambda i: (i, 0)),
    out_shape=jax.ShapeDtypeStruct((N, D), jnp.float32),
)


@jax.jit
def kernel(X, X_importance, edge_index, edge_type, index,
           comp0, bases0, root0, bias0, gamma0, beta0, mean0, var0,
           comp1, bases1, root1, bias1, gamma1, beta1, mean1, var1):
    del index
    epad = NCHP * CH - E
    pad_ids = jnp.arange(epad, dtype=jnp.int32)
    srcp = jnp.concatenate([edge_index[0], pad_ids % N])
    dstp = jnp.concatenate([edge_index[1], pad_ids % N])
    etp = jnp.concatenate([edge_type, pad_ids % R])

    cnt = _cnt_kernel(dstp, etp).reshape(NW, N, R)
    x0, y0, inv = _prep_kernel(X, X_importance, cnt, comp0, bases0)
    invp = _inv_kernel(dstp, etp, inv.reshape(N * R))

    aggp0 = _edge_kernel(y0.reshape(N * R, D), srcp, dstp, etp,
                         invp).reshape(NC, N, D)
    x1, y1 = _mid_kernel(aggp0, x0, root0, bias0.reshape(1, D),
                         gamma0.reshape(1, D), beta0.reshape(1, D),
                         mean0.reshape(1, D), var0.reshape(1, D),
                         comp1, bases1)
    aggp1 = _edge_kernel(y1.reshape(N * R, D), srcp, dstp, etp,
                         invp).reshape(NC, N, D)
    out = _final_kernel(aggp1, x1, root1, bias1.reshape(1, D),
                        gamma1.reshape(1, D), beta1.reshape(1, D),
                        mean1.reshape(1, D), var1.reshape(1, D))
    return out
